# baseline (speedup 1.0000x reference)
import functools

import jax
import jax.numpy as jnp
from jax import lax
from jax.experimental import pallas as pl
from jax.experimental.pallas import tpu as pltpu
from jax.experimental.pallas import tpu_sc as plsc

BATCH = 16384
HID = 64
NC = 2
NS = 16
NW = NC * NS
BPW = BATCH // NW
CHUNK = 128
NCHUNK = BPW // CHUNK

def _make_sc_gather():
    mesh = plsc.VectorSubcoreMesh(core_axis_name="c", subcore_axis_name="s")

    @functools.partial(
        pl.kernel,
        out_type=[
            jax.ShapeDtypeStruct((NW, NCHUNK, CHUNK, HID), jnp.float32),
            jax.ShapeDtypeStruct((NW, NCHUNK, CHUNK, HID), jnp.float32),
        ],
        mesh=mesh,
        scratch_types=[
            pltpu.VMEM((NCHUNK, CHUNK), jnp.int32),
            pltpu.VMEM((NCHUNK, CHUNK), jnp.int32),
            pltpu.VMEM((NCHUNK, CHUNK, HID), jnp.float32),
            pltpu.VMEM((NCHUNK, CHUNK, HID), jnp.float32),
            pltpu.SemaphoreType.DMA,
        ],
        compiler_params=pltpu.CompilerParams(use_tc_tiling_on_sc=False),
    )
    def _sc_gather(sidx_hbm, qidx_hbm, stab_hbm, qtab_hbm, s_out, q_out,
                   sidx_v, qidx_v, srows_v, qrows_v, sem):
        wid = lax.axis_index("s") * NC + lax.axis_index("c")
        pltpu.sync_copy(sidx_hbm.at[wid], sidx_v)
        pltpu.sync_copy(qidx_hbm.at[wid], qidx_v)
        copies = []
        for c in range(NCHUNK):
            copies.append(pltpu.async_copy(stab_hbm.at[sidx_v.at[c]], srows_v.at[c], sem))
            copies.append(pltpu.async_copy(qtab_hbm.at[qidx_v.at[c]], qrows_v.at[c], sem))
        for cp in copies:
            cp.wait()
        pltpu.sync_copy(srows_v, s_out.at[wid])
        pltpu.sync_copy(qrows_v, q_out.at[wid])

    return _sc_gather


BS = 2048


def _mlp_body(s_ref, q_ref, ts_ref, tt_ref, w1s_ref, w1q_ref, w1t_ref,
              b1_ref, w2_ref, b2_ref, w3_ref, b3_ref, out_ref):
    s = s_ref[...]
    q = q_ref[...]
    ts = ts_ref[...]
    binned = jnp.clip(ts // 60, 0, 9)
    oh = (binned == lax.broadcasted_iota(jnp.int32, (1, 16), 1)
          ).astype(jnp.float32)
    ttp = jnp.dot(tt_ref[...], w1t_ref[...],
                  preferred_element_type=jnp.float32)
    x1 = (jnp.dot(s, w1s_ref[...], preferred_element_type=jnp.float32)
          + jnp.dot(q, w1q_ref[...], preferred_element_type=jnp.float32)
          + jnp.dot(oh, ttp, preferred_element_type=jnp.float32)
          + b1_ref[...])
    h1 = jnp.maximum(x1, 0.0)
    h2 = jnp.maximum(
        jnp.dot(h1, w2_ref[...], preferred_element_type=jnp.float32)
        + b2_ref[...], 0.0)
    o = jnp.sum(h2 * w3_ref[...], axis=1, keepdims=True) + b3_ref[...]
    out_ref[...] = jax.nn.sigmoid(o)


def _mlp_call(s_g, q_g, ts2, tt16, w1s, w1q, w1t, b1r, w2, b2r, w3r, b3r):
    grid = (BATCH // BS,)
    return pl.pallas_call(
        _mlp_body,
        grid=grid,
        in_specs=[
            pl.BlockSpec((BS, HID), lambda i: (i, 0)),
            pl.BlockSpec((BS, HID), lambda i: (i, 0)),
            pl.BlockSpec((BS, 1), lambda i: (i, 0)),
            pl.BlockSpec((16, 32), lambda i: (0, 0)),
            pl.BlockSpec((HID, 128), lambda i: (0, 0)),
            pl.BlockSpec((HID, 128), lambda i: (0, 0)),
            pl.BlockSpec((32, 128), lambda i: (0, 0)),
            pl.BlockSpec((1, 128), lambda i: (0, 0)),
            pl.BlockSpec((128, HID), lambda i: (0, 0)),
            pl.BlockSpec((1, HID), lambda i: (0, 0)),
            pl.BlockSpec((1, HID), lambda i: (0, 0)),
            pl.BlockSpec((1, 1), lambda i: (0, 0)),
        ],
        out_specs=pl.BlockSpec((BS, 1), lambda i: (i, 0)),
        out_shape=jax.ShapeDtypeStruct((BATCH, 1), jnp.float32),
    )(s_g, q_g, ts2, tt16, w1s, w1q, w1t, b1r, w2, b2r, w3r, b3r)


def kernel(s_ids, i_ids, time_spent, student_table, item_table, time_table,
           W1, b1, W2, b2, W3, b3):
    sidx = s_ids.reshape(NW, NCHUNK, CHUNK)
    qidx = i_ids.reshape(NW, NCHUNK, CHUNK)
    s_g, q_g = _make_sc_gather()(sidx, qidx, student_table, item_table)
    s_g = s_g.reshape(BATCH, HID)
    q_g = q_g.reshape(BATCH, HID)

    tt16 = jnp.zeros((16, 32), jnp.float32).at[:10].set(time_table)
    out = _mlp_call(
        s_g, q_g,
        time_spent.reshape(BATCH, 1),
        tt16,
        W1[:HID], W1[HID:2 * HID], W1[2 * HID:],
        b1.reshape(1, 128),
        W2,
        b2.reshape(1, HID),
        W3.reshape(1, HID),
        b3.reshape(1, 1),
    )
    return out


# speedup vs baseline: 1.2720x; 1.2720x over previous
import functools

import jax
import jax.numpy as jnp
from jax import lax
from jax.experimental import pallas as pl
from jax.experimental.pallas import tpu as pltpu
from jax.experimental.pallas import tpu_sc as plsc

BATCH = 16384
HID = 64
NC = 2
NS = 16
NW = NC * NS
BPW = BATCH // NW
K = 16
CH = 128
NBATCH = CH // K
NCH = BPW // CH


def _make_sc_gather():
    mesh = plsc.VectorSubcoreMesh(core_axis_name="c", subcore_axis_name="s")

    @functools.partial(
        pl.kernel,
        out_type=[
            jax.ShapeDtypeStruct((BATCH, HID), jnp.float32),
            jax.ShapeDtypeStruct((BATCH, HID), jnp.float32),
        ],
        mesh=mesh,
        scratch_types=[
            pltpu.VMEM((BPW,), jnp.int32),
            pltpu.VMEM((BPW,), jnp.int32),
            pltpu.VMEM((K, 16, HID), jnp.float32),
            pltpu.VMEM((K, 16, HID), jnp.float32),
            pltpu.VMEM((CH, HID), jnp.float32),
            pltpu.VMEM((CH, HID), jnp.float32),
            pltpu.SemaphoreType.DMA,
        ],
        compiler_params=pltpu.CompilerParams(use_tc_tiling_on_sc=True),
    )
    def _sc_gather(sidx_hbm, qidx_hbm, stab_hbm, qtab_hbm, s_out, q_out,
                   sidx_v, qidx_v, sstage_v, qstage_v, srows_v, qrows_v, sem):
        wid = lax.axis_index("s") * NC + lax.axis_index("c")
        base = wid * BPW
        pltpu.sync_copy(sidx_hbm.at[pl.ds(base, BPW)], sidx_v)
        pltpu.sync_copy(qidx_hbm.at[pl.ds(base, BPW)], qidx_v)

        def chunk_body(c, _):
            for b in range(NBATCH):
                off = c * CH + b * K
                sv = sidx_v[pl.ds(off, K)]
                qv = qidx_v[pl.ds(off, K)]
                copies = []
                srr = []
                qrr = []
                for j in range(K):
                    r = sv[j]
                    r16 = pl.multiple_of((r >> 4) << 4, 16)
                    srr.append(r & 15)
                    copies.append(pltpu.async_copy(
                        stab_hbm.at[pl.ds(r16, 16)], sstage_v.at[j], sem))
                    rq = qv[j]
                    rq16 = pl.multiple_of((rq >> 4) << 4, 16)
                    qrr.append(rq & 15)
                    copies.append(pltpu.async_copy(
                        qtab_hbm.at[pl.ds(rq16, 16)], qstage_v.at[j], sem))
                for cp in copies:
                    cp.wait()
                for j in range(K):
                    ci = b * K + j
                    for jj in range(HID // 16):
                        srows_v[ci, pl.ds(jj * 16, 16)] = (
                            sstage_v[j, srr[j], pl.ds(jj * 16, 16)])
                        qrows_v[ci, pl.ds(jj * 16, 16)] = (
                            qstage_v[j, qrr[j], pl.ds(jj * 16, 16)])
            flush = pl.multiple_of(base + c * CH, CH)
            pltpu.sync_copy(srows_v, s_out.at[pl.ds(flush, CH)])
            pltpu.sync_copy(qrows_v, q_out.at[pl.ds(flush, CH)])
            return ()

        lax.fori_loop(0, NCH, chunk_body, (), unroll=False)

    return _sc_gather


BS = 2048


def _mlp_body(s_ref, q_ref, ts_ref, tt_ref, w1s_ref, w1q_ref, w1t_ref,
              b1_ref, w2_ref, b2_ref, w3_ref, b3_ref, out_ref):
    s = s_ref[...]
    q = q_ref[...]
    ts = ts_ref[...]
    binned = jnp.clip(ts // 60, 0, 9)
    oh = (binned == lax.broadcasted_iota(jnp.int32, (1, 16), 1)
          ).astype(jnp.float32)
    ttp = jnp.dot(tt_ref[...], w1t_ref[...],
                  preferred_element_type=jnp.float32)
    x1 = (jnp.dot(s, w1s_ref[...], preferred_element_type=jnp.float32)
          + jnp.dot(q, w1q_ref[...], preferred_element_type=jnp.float32)
          + jnp.dot(oh, ttp, preferred_element_type=jnp.float32)
          + b1_ref[...])
    h1 = jnp.maximum(x1, 0.0)
    h2 = jnp.maximum(
        jnp.dot(h1, w2_ref[...], preferred_element_type=jnp.float32)
        + b2_ref[...], 0.0)
    o = jnp.sum(h2 * w3_ref[...], axis=1, keepdims=True) + b3_ref[...]
    out_ref[...] = jax.nn.sigmoid(o)


def _mlp_call(s_g, q_g, ts2, tt16, w1s, w1q, w1t, b1r, w2, b2r, w3r, b3r):
    grid = (BATCH // BS,)
    return pl.pallas_call(
        _mlp_body,
        grid=grid,
        in_specs=[
            pl.BlockSpec((BS, HID), lambda i: (i, 0)),
            pl.BlockSpec((BS, HID), lambda i: (i, 0)),
            pl.BlockSpec((BS, 1), lambda i: (i, 0)),
            pl.BlockSpec((16, 32), lambda i: (0, 0)),
            pl.BlockSpec((HID, 128), lambda i: (0, 0)),
            pl.BlockSpec((HID, 128), lambda i: (0, 0)),
            pl.BlockSpec((32, 128), lambda i: (0, 0)),
            pl.BlockSpec((1, 128), lambda i: (0, 0)),
            pl.BlockSpec((128, HID), lambda i: (0, 0)),
            pl.BlockSpec((1, HID), lambda i: (0, 0)),
            pl.BlockSpec((1, HID), lambda i: (0, 0)),
            pl.BlockSpec((1, 1), lambda i: (0, 0)),
        ],
        out_specs=pl.BlockSpec((BS, 1), lambda i: (i, 0)),
        out_shape=jax.ShapeDtypeStruct((BATCH, 1), jnp.float32),
    )(s_g, q_g, ts2, tt16, w1s, w1q, w1t, b1r, w2, b2r, w3r, b3r)


def kernel(s_ids, i_ids, time_spent, student_table, item_table, time_table,
           W1, b1, W2, b2, W3, b3):
    s_g, q_g = _make_sc_gather()(s_ids, i_ids, student_table, item_table)

    tt16 = jnp.zeros((16, 32), jnp.float32).at[:10].set(time_table)
    out = _mlp_call(
        s_g, q_g,
        time_spent.reshape(BATCH, 1),
        tt16,
        W1[:HID], W1[HID:2 * HID], W1[2 * HID:],
        b1.reshape(1, 128),
        W2,
        b2.reshape(1, HID),
        W3.reshape(1, HID),
        b3.reshape(1, 1),
    )
    return out
